# Initial kernel scaffold; baseline (speedup 1.0000x reference)
import jax
import jax.numpy as jnp
from jax.experimental import pallas as pl


def kernel(x, h_prev, c_prev, edge_index, W_gat, att_src, att_dst, b_gat, W_ih, b_ih, W_hh, b_hh):
    raise NotImplementedError("write your pallas kernel here")


# speedup vs baseline: 23.2166x; 23.2166x over previous
import functools

import jax
import jax.numpy as jnp
from jax import lax
from jax.experimental import pallas as pl
from jax.experimental.pallas import tpu as pltpu
from jax.experimental.pallas import tpu_sc as plsc

NC = 2
NS = 16
LN = 16
HW = 32

BN = 2000
EB = 400
SUB = 80
NSUB = EB // SUB


def _prep_body(x_ref, h_ref, w1_ref, w2_ref, as_ref, ad_ref,
               xp_ref, asrc_ref, adst_ref, cmax_ref, acc_ref):
    i = pl.program_id(0)
    xp = (jnp.dot(x_ref[...], w1_ref[...], preferred_element_type=jnp.float32)
          + jnp.dot(h_ref[...], w2_ref[...], preferred_element_type=jnp.float32))
    a_s = jnp.dot(xp, as_ref[...], preferred_element_type=jnp.float32)
    a_d = jnp.dot(xp, ad_ref[...], preferred_element_type=jnp.float32)
    xp_ref[...] = xp
    asrc_ref[...] = a_s
    adst_ref[...] = a_d

    @pl.when(i == 0)
    def _():
        acc_ref[0] = jnp.float32(-1e30)
        acc_ref[1] = jnp.float32(-1e30)

    acc_ref[0] = jnp.maximum(acc_ref[0], jnp.max(a_s))
    acc_ref[1] = jnp.maximum(acc_ref[1], jnp.max(a_d))
    cmax_ref[...] = jnp.full((1, 1), jnp.maximum(acc_ref[0] + acc_ref[1],
                                                 jnp.float32(0.0)), jnp.float32)


def _sc_scalar_body(src_hbm, dst_hbm, asrc_hbm, adst_hbm, c_hbm, zd_hbm,
                    w_hbm, den_hbm,
                    asrc_tbl, adst_tbl, cbuf, srcbuf, dst2, wbuf, den_acc):
    n = asrc_tbl.shape[0]
    ept = src_hbm.shape[0] // NS
    nch = ept // EB
    cid = lax.axis_index("c")
    tid = lax.axis_index("s")
    half = (nch + 1) // 2
    klo = jnp.where(cid == 0, 0, half)
    khi = jnp.where(cid == 0, half, nch)

    pltpu.sync_copy(asrc_hbm, asrc_tbl)
    pltpu.sync_copy(adst_hbm, adst_tbl)
    pltpu.sync_copy(c_hbm, cbuf)
    cvec = cbuf[...]

    @pl.when(tid == 0)
    def _():
        pltpu.sync_copy(zd_hbm, den_acc)

    plsc.subcore_barrier()

    def chunk(k, carry):
        e0 = tid * ept + k * EB
        pltpu.sync_copy(src_hbm.at[pl.ds(e0, EB)], srcbuf)

        def dget(j, c):
            pltpu.sync_copy(dst_hbm.at[pl.ds(e0 + j * SUB, SUB)], dst2.at[j])
            return c

        lax.fori_loop(0, NSUB, dget, 0)
        for j in range(NSUB):
            for m in range(SUB // LN):
                off = j * SUB + m * LN
                sv = srcbuf[pl.ds(off, LN)]
                dv = dst2[j, pl.ds(m * LN, LN)]
                av = plsc.load_gather(asrc_tbl, [sv])
                bv = plsc.load_gather(adst_tbl, [dv])
                e = av + bv
                e = jnp.where(e >= 0, e, jnp.float32(0.2) * e)
                wbuf[pl.ds(off, LN)] = jnp.exp(e - cvec)
        pltpu.sync_copy(wbuf, w_hbm.at[pl.ds(e0, EB)])

        def scat(j, c):
            pltpu.sync_copy(wbuf.at[pl.ds(j * SUB, SUB)],
                            den_acc.at[dst2.at[j]], add=True)
            return c

        lax.fori_loop(0, NSUB, scat, 0)
        return carry

    lax.fori_loop(klo, khi, chunk, 0)
    plsc.subcore_barrier()

    @pl.when(tid == 0)
    def _():
        pltpu.sync_copy(den_acc, den_hbm.at[pl.ds(cid * n, n)])


def _sc_row_body(src_hbm, dst_hbm, w_hbm, xp2_hbm, zn_hbm, num_hbm,
                 srcbuf, dst2, wbuf, rows, num_acc, sem_g):
    n = num_acc.shape[0]
    ept = src_hbm.shape[0] // NS
    nch = ept // EB
    cid = lax.axis_index("c")
    tid = lax.axis_index("s")
    coff = (cid * n).astype(jnp.int32)

    rpt8 = ((n // NS + 7) // 8) * 8
    last = n - (NS - 1) * rpt8
    assert last > 0 and last % 8 == 0

    @pl.when(tid < NS - 1)
    def _():
        pltpu.sync_copy(zn_hbm.at[pl.ds(tid * rpt8, rpt8)],
                        num_acc.at[pl.ds(tid * rpt8, rpt8)])

    @pl.when(tid == NS - 1)
    def _():
        pltpu.sync_copy(zn_hbm.at[pl.ds((NS - 1) * rpt8, last)],
                        num_acc.at[pl.ds((NS - 1) * rpt8, last)])

    plsc.subcore_barrier()

    def chunk(k, carry):
        e0 = tid * ept + k * EB
        pltpu.sync_copy(src_hbm.at[pl.ds(e0, EB)], srcbuf)
        pltpu.sync_copy(w_hbm.at[pl.ds(e0, EB)], wbuf)

        def dget(j, c):
            pltpu.sync_copy(dst_hbm.at[pl.ds(e0 + j * SUB, SUB)], dst2.at[j])
            return c

        lax.fori_loop(0, NSUB, dget, 0)

        def shift(gi, c):
            off = gi * LN
            srcbuf[pl.ds(off, LN)] = srcbuf[pl.ds(off, LN)] + coff
            return c

        lax.fori_loop(0, EB // LN, shift, 0)
        descs = [
            pltpu.async_copy(
                xp2_hbm.at[srcbuf.at[pl.ds(j * SUB, SUB)]],
                rows.at[pl.ds(j * SUB, SUB)], sem_g)
            for j in range(NSUB)
        ]
        for d in descs:
            d.wait()

        def scale(gi, c):
            wv = wbuf[pl.ds(gi * LN, LN)]
            for l in range(LN):
                i = gi * LN + l
                wl = jnp.full((LN,), wv[l], jnp.float32)
                rows[i, pl.ds(0, LN)] = rows[i, pl.ds(0, LN)] * wl
                rows[i, pl.ds(LN, LN)] = rows[i, pl.ds(LN, LN)] * wl
            return c

        lax.fori_loop(0, EB // LN, scale, 0)

        def scat(j, c):
            pltpu.sync_copy(rows.at[pl.ds(j * SUB, SUB)],
                            num_acc.at[dst2.at[j]], add=True)
            return c

        lax.fori_loop(0, NSUB, scat, 0)
        return carry

    lax.fori_loop(0, nch, chunk, 0)
    plsc.subcore_barrier()

    @pl.when(tid < NS - 1)
    def _():
        pltpu.sync_copy(num_acc.at[pl.ds(tid * rpt8, rpt8)],
                        num_hbm.at[pl.ds(cid * n + tid * rpt8, rpt8)])

    @pl.when(tid == NS - 1)
    def _():
        pltpu.sync_copy(num_acc.at[pl.ds((NS - 1) * rpt8, last)],
                        num_hbm.at[pl.ds(cid * n + (NS - 1) * rpt8, last)])


def _finish_body(numl_ref, numr_ref, den0_ref, den1_ref, xp_ref, asrc_ref,
                 adst_ref, cmax_ref, h_ref, c_ref, wih_ref, whh_ref, bih_ref,
                 bhh_ref, bgat_ref, h_out, c_out):
    a = asrc_ref[...] + adst_ref[...]
    a = jnp.where(a >= 0, a, jnp.float32(0.2) * a)
    ws = jnp.exp(a - cmax_ref[...])
    num = jnp.concatenate([numl_ref[...], numr_ref[...]], axis=1)
    den = den0_ref[...] + den1_ref[...] + ws + jnp.float32(1e-16)
    hagg = (num + ws * xp_ref[...]) / den + bgat_ref[...]
    dn = (((1,), (1,)), ((), ()))
    gates = (lax.dot_general(hagg, wih_ref[...], dn,
                             preferred_element_type=jnp.float32)
             + lax.dot_general(h_ref[...], whh_ref[...], dn,
                               preferred_element_type=jnp.float32)
             + bih_ref[...] + bhh_ref[...])
    dh = h_ref.shape[1]
    ig = jax.nn.sigmoid(gates[:, 0:dh])
    fg = jax.nn.sigmoid(gates[:, dh:2 * dh])
    gg = jnp.tanh(gates[:, 2 * dh:3 * dh])
    og = jax.nn.sigmoid(gates[:, 3 * dh:4 * dh])
    cn = fg * c_ref[...] + ig * gg
    h_out[...] = og * jnp.tanh(cn)
    c_out[...] = cn


def kernel(x, h_prev, c_prev, edge_index, W_gat, att_src, att_dst, b_gat,
           W_ih, b_ih, W_hh, b_hh):
    n, d_in = x.shape
    dh = h_prev.shape[1]
    e = edge_index.shape[1]
    g = n // BN
    assert n % BN == 0 and n % NS == 0 and dh == 2 * HW
    assert e % (NS * EB) == 0

    f32 = jnp.float32
    w1 = W_gat[:d_in]
    w2 = W_gat[d_in:]
    as2 = att_src.reshape(dh, 1)
    ad2 = att_dst.reshape(dh, 1)

    xp, asrc, adst, cmax = pl.pallas_call(
        _prep_body,
        grid=(g,),
        in_specs=[
            pl.BlockSpec((BN, d_in), lambda i: (i, 0)),
            pl.BlockSpec((BN, dh), lambda i: (i, 0)),
            pl.BlockSpec((d_in, dh), lambda i: (0, 0)),
            pl.BlockSpec((dh, dh), lambda i: (0, 0)),
            pl.BlockSpec((dh, 1), lambda i: (0, 0)),
            pl.BlockSpec((dh, 1), lambda i: (0, 0)),
        ],
        out_specs=[
            pl.BlockSpec((BN, dh), lambda i: (i, 0)),
            pl.BlockSpec((BN, 1), lambda i: (i, 0)),
            pl.BlockSpec((BN, 1), lambda i: (i, 0)),
            pl.BlockSpec((1, 1), lambda i: (0, 0)),
        ],
        out_shape=[
            jax.ShapeDtypeStruct((n, dh), f32),
            jax.ShapeDtypeStruct((n, 1), f32),
            jax.ShapeDtypeStruct((n, 1), f32),
            jax.ShapeDtypeStruct((1, 1), f32),
        ],
        scratch_shapes=[pltpu.SMEM((2,), f32)],
    )(x, h_prev, w1, w2, as2, ad2)

    xp2 = jnp.concatenate([xp[:, 0:HW], xp[:, HW:2 * HW]], axis=0)
    src = edge_index[0]
    dst = edge_index[1]
    cvec = jnp.broadcast_to(cmax.reshape(1), (LN,))
    zn = jnp.zeros((n, HW), f32)
    zd = jnp.zeros((n,), f32)

    mesh = plsc.VectorSubcoreMesh(core_axis_name="c", subcore_axis_name="s",
                                  num_cores=NC, num_subcores=NS)
    scp = pltpu.CompilerParams(needs_layout_passes=False,
                               use_tc_tiling_on_sc=False)
    wvals, den2 = pl.kernel(
        _sc_scalar_body,
        out_type=[
            jax.ShapeDtypeStruct((e,), f32),
            jax.ShapeDtypeStruct((2 * n,), f32),
        ],
        mesh=mesh,
        compiler_params=scp,
        scratch_types=[
            pltpu.VMEM((n,), f32),
            pltpu.VMEM((n,), f32),
            pltpu.VMEM((LN,), f32),
            pltpu.VMEM((EB,), jnp.int32),
            pltpu.VMEM((NSUB, SUB), jnp.int32),
            pltpu.VMEM((EB,), f32),
            pltpu.VMEM_SHARED((n,), f32),
        ],
    )(src, dst, asrc.reshape(n), adst.reshape(n), cvec, zd)

    num2 = pl.kernel(
        _sc_row_body,
        out_type=jax.ShapeDtypeStruct((2 * n, HW), f32),
        mesh=mesh,
        compiler_params=scp,
        scratch_types=[
            pltpu.VMEM((EB,), jnp.int32),
            pltpu.VMEM((NSUB, SUB), jnp.int32),
            pltpu.VMEM((EB,), f32),
            pltpu.VMEM((EB, HW), f32),
            pltpu.VMEM_SHARED((n, HW), f32),
            pltpu.SemaphoreType.DMA,
        ],
    )(src, dst, wvals, xp2, zn)

    h_next, c_next = pl.pallas_call(
        _finish_body,
        grid=(g,),
        in_specs=[
            pl.BlockSpec((BN, HW), lambda i: (i, 0)),
            pl.BlockSpec((BN, HW), lambda i: (g + i, 0)),
            pl.BlockSpec((BN, 1), lambda i: (i, 0)),
            pl.BlockSpec((BN, 1), lambda i: (g + i, 0)),
            pl.BlockSpec((BN, dh), lambda i: (i, 0)),
            pl.BlockSpec((BN, 1), lambda i: (i, 0)),
            pl.BlockSpec((BN, 1), lambda i: (i, 0)),
            pl.BlockSpec((1, 1), lambda i: (0, 0)),
            pl.BlockSpec((BN, dh), lambda i: (i, 0)),
            pl.BlockSpec((BN, dh), lambda i: (i, 0)),
            pl.BlockSpec((4 * dh, dh), lambda i: (0, 0)),
            pl.BlockSpec((4 * dh, dh), lambda i: (0, 0)),
            pl.BlockSpec((1, 4 * dh), lambda i: (0, 0)),
            pl.BlockSpec((1, 4 * dh), lambda i: (0, 0)),
            pl.BlockSpec((1, dh), lambda i: (0, 0)),
        ],
        out_specs=[
            pl.BlockSpec((BN, dh), lambda i: (i, 0)),
            pl.BlockSpec((BN, dh), lambda i: (i, 0)),
        ],
        out_shape=[
            jax.ShapeDtypeStruct((n, dh), f32),
            jax.ShapeDtypeStruct((n, dh), f32),
        ],
    )(num2, num2, den2.reshape(2 * n, 1), den2.reshape(2 * n, 1), xp, asrc,
      adst, cmax, h_prev, c_prev, W_ih, W_hh, b_ih.reshape(1, 4 * dh),
      b_hh.reshape(1, 4 * dh), b_gat.reshape(1, dh))
    return (h_next, c_next)


# speedup vs baseline: 32.3719x; 1.3943x over previous
import functools

import jax
import jax.numpy as jnp
from jax import lax
from jax.experimental import pallas as pl
from jax.experimental.pallas import tpu as pltpu
from jax.experimental.pallas import tpu_sc as plsc

NC = 2
NS = 16
LN = 16
HW = 32

BN = 2000
EB = 400
SUB = 80
NSUB = EB // SUB


def _prep_body(x_ref, h_ref, w1_ref, w2_ref, as_ref, ad_ref,
               xp_ref, asrc_ref, adst_ref, cmax_ref, acc_ref):
    i = pl.program_id(0)
    xp = (jnp.dot(x_ref[...], w1_ref[...], preferred_element_type=jnp.float32)
          + jnp.dot(h_ref[...], w2_ref[...], preferred_element_type=jnp.float32))
    a_s = jnp.dot(xp, as_ref[...], preferred_element_type=jnp.float32)
    a_d = jnp.dot(xp, ad_ref[...], preferred_element_type=jnp.float32)
    xp_ref[...] = xp
    asrc_ref[...] = a_s
    adst_ref[...] = a_d

    @pl.when(i == 0)
    def _():
        acc_ref[0] = jnp.float32(-1e30)
        acc_ref[1] = jnp.float32(-1e30)

    acc_ref[0] = jnp.maximum(acc_ref[0], jnp.max(a_s))
    acc_ref[1] = jnp.maximum(acc_ref[1], jnp.max(a_d))
    cmax_ref[...] = jnp.full((1, 1), jnp.maximum(acc_ref[0] + acc_ref[1],
                                                 jnp.float32(0.0)), jnp.float32)


def _sc_scalar_body(src_hbm, dst_hbm, asrc_hbm, adst_hbm, c_hbm, zd_hbm,
                    w_hbm, den_hbm,
                    asrc_tbl, adst_tbl, cbuf, srcbuf, dst2, wbuf, den_acc):
    n = asrc_tbl.shape[0]
    ept = src_hbm.shape[0] // NS
    nch = ept // EB
    cid = lax.axis_index("c")
    tid = lax.axis_index("s")
    half = (nch + 1) // 2
    klo = jnp.where(cid == 0, 0, half)
    khi = jnp.where(cid == 0, half, nch)

    pltpu.sync_copy(asrc_hbm, asrc_tbl)
    pltpu.sync_copy(adst_hbm, adst_tbl)
    pltpu.sync_copy(c_hbm, cbuf)
    cvec = cbuf[...]

    @pl.when(tid == 0)
    def _():
        pltpu.sync_copy(zd_hbm, den_acc)

    plsc.subcore_barrier()

    def chunk(k, carry):
        e0 = tid * ept + k * EB
        pltpu.sync_copy(src_hbm.at[pl.ds(e0, EB)], srcbuf)

        def dget(j, c):
            pltpu.sync_copy(dst_hbm.at[pl.ds(e0 + j * SUB, SUB)], dst2.at[j])
            return c

        lax.fori_loop(0, NSUB, dget, 0)
        for j in range(NSUB):
            for m in range(SUB // LN):
                off = j * SUB + m * LN
                sv = srcbuf[pl.ds(off, LN)]
                dv = dst2[j, pl.ds(m * LN, LN)]
                av = plsc.load_gather(asrc_tbl, [sv])
                bv = plsc.load_gather(adst_tbl, [dv])
                e = av + bv
                e = jnp.where(e >= 0, e, jnp.float32(0.2) * e)
                wbuf[pl.ds(off, LN)] = jnp.exp(e - cvec)
        pltpu.sync_copy(wbuf, w_hbm.at[pl.ds(e0, EB)])

        def scat(j, c):
            pltpu.sync_copy(wbuf.at[pl.ds(j * SUB, SUB)],
                            den_acc.at[dst2.at[j]], add=True)
            return c

        lax.fori_loop(0, NSUB, scat, 0)
        return carry

    lax.fori_loop(klo, khi, chunk, 0)
    plsc.subcore_barrier()

    @pl.when(tid == 0)
    def _():
        pltpu.sync_copy(den_acc, den_hbm.at[pl.ds(cid * n, n)])


def _sc_row_body(src_hbm, dst_hbm, w_hbm, xp2_hbm, zn_hbm, num_hbm,
                 srcbuf, dst2, sdst2, wbuf, rows, num_acc,
                 sem_i0, sem_i1, sem_g0, sem_g1, sem_s0, sem_s1):
    n = num_acc.shape[0]
    ept = src_hbm.shape[0] // NS
    nch = ept // EB
    cid = lax.axis_index("c")
    tid = lax.axis_index("s")
    coff = (cid * n).astype(jnp.int32)
    sem_i = (sem_i0, sem_i1)
    sem_g = (sem_g0, sem_g1)
    sem_s = (sem_s0, sem_s1)

    rpt8 = ((n // NS + 7) // 8) * 8
    last = n - (NS - 1) * rpt8
    assert last > 0 and last % 8 == 0

    def idx_trips(k, b):
        e0 = tid * ept + k * EB
        t = [(src_hbm.at[pl.ds(e0, EB)], srcbuf.at[b]),
             (w_hbm.at[pl.ds(e0, EB)], wbuf.at[b])]
        t += [(dst_hbm.at[pl.ds(e0 + j * SUB, SUB)], dst2.at[b, j])
              for j in range(NSUB)]
        return t

    def issue_idx(k, b):
        for s, d in idx_trips(k, b):
            pltpu.async_copy(s, d, sem_i[b])

    def wait_idx(k, b):
        for s, d in idx_trips(k, b):
            pltpu.make_async_copy(s, d, sem_i[b]).wait()

    def drain_scat(b):
        for j in range(NSUB):
            pltpu.make_async_copy(rows.at[b, pl.ds(j * SUB, SUB)],
                                  num_acc.at[sdst2.at[b, j]], sem_s[b]).wait()

    def fire_gather(j, b):
        pltpu.async_copy(xp2_hbm.at[srcbuf.at[b, pl.ds(j * SUB, SUB)]],
                         rows.at[b, pl.ds(j * SUB, SUB)], sem_g[j % 2])

    def drain_gather(j, b):
        pltpu.make_async_copy(xp2_hbm.at[srcbuf.at[b, pl.ds(j * SUB, SUB)]],
                              rows.at[b, pl.ds(j * SUB, SUB)],
                              sem_g[j % 2]).wait()

    def process(k, b):
        @pl.when(k >= 2)
        def _():
            drain_scat(b)

        wait_idx(k, b)

        def shift(gi, c):
            off = gi * LN
            srcbuf[b, pl.ds(off, LN)] = srcbuf[b, pl.ds(off, LN)] + coff
            return c

        lax.fori_loop(0, EB // LN, shift, 0)
        for j in range(NSUB):
            for m in range(SUB // LN):
                sdst2[b, j, pl.ds(m * LN, LN)] = dst2[b, j, pl.ds(m * LN, LN)]

        def scale(gi, c):
            wv = wbuf[b, pl.ds(gi * LN, LN)]
            for l in range(LN):
                i = gi * LN + l
                wl = jnp.full((LN,), wv[l], jnp.float32)
                rows[b, i, pl.ds(0, LN)] = rows[b, i, pl.ds(0, LN)] * wl
                rows[b, i, pl.ds(LN, LN)] = rows[b, i, pl.ds(LN, LN)] * wl
            return c

        fire_gather(0, b)
        for j in range(NSUB):
            if j + 1 < NSUB:
                fire_gather(j + 1, b)
            drain_gather(j, b)
            lax.fori_loop(j * (SUB // LN), (j + 1) * (SUB // LN), scale, 0)
            pltpu.async_copy(rows.at[b, pl.ds(j * SUB, SUB)],
                             num_acc.at[sdst2.at[b, j]], sem_s[b], add=True)

        @pl.when(k + 2 < nch)
        def _():
            issue_idx(k + 2, b)

    issue_idx(0, 0)
    issue_idx(1, 1)

    @pl.when(tid < NS - 1)
    def _():
        pltpu.sync_copy(zn_hbm.at[pl.ds(tid * rpt8, rpt8)],
                        num_acc.at[pl.ds(tid * rpt8, rpt8)])

    @pl.when(tid == NS - 1)
    def _():
        pltpu.sync_copy(zn_hbm.at[pl.ds((NS - 1) * rpt8, last)],
                        num_acc.at[pl.ds((NS - 1) * rpt8, last)])

    plsc.subcore_barrier()

    def loop2(k2, c):
        process(2 * k2, 0)
        process(2 * k2 + 1, 1)
        return c

    lax.fori_loop(0, nch // 2, loop2, 0)
    for k in range(2 * (nch // 2), nch):
        process(jnp.int32(k), k % 2)
    for b in range(2):
        drain_scat(b)
    plsc.subcore_barrier()

    @pl.when(tid < NS - 1)
    def _():
        pltpu.sync_copy(num_acc.at[pl.ds(tid * rpt8, rpt8)],
                        num_hbm.at[pl.ds(cid * n + tid * rpt8, rpt8)])

    @pl.when(tid == NS - 1)
    def _():
        pltpu.sync_copy(num_acc.at[pl.ds((NS - 1) * rpt8, last)],
                        num_hbm.at[pl.ds(cid * n + (NS - 1) * rpt8, last)])


def _finish_body(numl_ref, numr_ref, den0_ref, den1_ref, xp_ref, asrc_ref,
                 adst_ref, cmax_ref, h_ref, c_ref, wih_ref, whh_ref, bih_ref,
                 bhh_ref, bgat_ref, h_out, c_out):
    a = asrc_ref[...] + adst_ref[...]
    a = jnp.where(a >= 0, a, jnp.float32(0.2) * a)
    ws = jnp.exp(a - cmax_ref[...])
    num = jnp.concatenate([numl_ref[...], numr_ref[...]], axis=1)
    den = den0_ref[...] + den1_ref[...] + ws + jnp.float32(1e-16)
    hagg = (num + ws * xp_ref[...]) / den + bgat_ref[...]
    dn = (((1,), (1,)), ((), ()))
    gates = (lax.dot_general(hagg, wih_ref[...], dn,
                             preferred_element_type=jnp.float32)
             + lax.dot_general(h_ref[...], whh_ref[...], dn,
                               preferred_element_type=jnp.float32)
             + bih_ref[...] + bhh_ref[...])
    dh = h_ref.shape[1]
    ig = jax.nn.sigmoid(gates[:, 0:dh])
    fg = jax.nn.sigmoid(gates[:, dh:2 * dh])
    gg = jnp.tanh(gates[:, 2 * dh:3 * dh])
    og = jax.nn.sigmoid(gates[:, 3 * dh:4 * dh])
    cn = fg * c_ref[...] + ig * gg
    h_out[...] = og * jnp.tanh(cn)
    c_out[...] = cn


def kernel(x, h_prev, c_prev, edge_index, W_gat, att_src, att_dst, b_gat,
           W_ih, b_ih, W_hh, b_hh):
    n, d_in = x.shape
    dh = h_prev.shape[1]
    e = edge_index.shape[1]
    g = n // BN
    assert n % BN == 0 and n % NS == 0 and dh == 2 * HW
    assert e % (NS * EB) == 0

    f32 = jnp.float32
    w1 = W_gat[:d_in]
    w2 = W_gat[d_in:]
    as2 = att_src.reshape(dh, 1)
    ad2 = att_dst.reshape(dh, 1)

    xp, asrc, adst, cmax = pl.pallas_call(
        _prep_body,
        grid=(g,),
        in_specs=[
            pl.BlockSpec((BN, d_in), lambda i: (i, 0)),
            pl.BlockSpec((BN, dh), lambda i: (i, 0)),
            pl.BlockSpec((d_in, dh), lambda i: (0, 0)),
            pl.BlockSpec((dh, dh), lambda i: (0, 0)),
            pl.BlockSpec((dh, 1), lambda i: (0, 0)),
            pl.BlockSpec((dh, 1), lambda i: (0, 0)),
        ],
        out_specs=[
            pl.BlockSpec((BN, dh), lambda i: (i, 0)),
            pl.BlockSpec((BN, 1), lambda i: (i, 0)),
            pl.BlockSpec((BN, 1), lambda i: (i, 0)),
            pl.BlockSpec((1, 1), lambda i: (0, 0)),
        ],
        out_shape=[
            jax.ShapeDtypeStruct((n, dh), f32),
            jax.ShapeDtypeStruct((n, 1), f32),
            jax.ShapeDtypeStruct((n, 1), f32),
            jax.ShapeDtypeStruct((1, 1), f32),
        ],
        scratch_shapes=[pltpu.SMEM((2,), f32)],
    )(x, h_prev, w1, w2, as2, ad2)

    xp2 = jnp.concatenate([xp[:, 0:HW], xp[:, HW:2 * HW]], axis=0)
    src = edge_index[0]
    dst = edge_index[1]
    cvec = jnp.broadcast_to(cmax.reshape(1), (LN,))
    zn = jnp.zeros((n, HW), f32)
    zd = jnp.zeros((n,), f32)

    mesh = plsc.VectorSubcoreMesh(core_axis_name="c", subcore_axis_name="s",
                                  num_cores=NC, num_subcores=NS)
    scp = pltpu.CompilerParams(needs_layout_passes=False,
                               use_tc_tiling_on_sc=False)
    wvals, den2 = pl.kernel(
        _sc_scalar_body,
        out_type=[
            jax.ShapeDtypeStruct((e,), f32),
            jax.ShapeDtypeStruct((2 * n,), f32),
        ],
        mesh=mesh,
        compiler_params=scp,
        scratch_types=[
            pltpu.VMEM((n,), f32),
            pltpu.VMEM((n,), f32),
            pltpu.VMEM((LN,), f32),
            pltpu.VMEM((EB,), jnp.int32),
            pltpu.VMEM((NSUB, SUB), jnp.int32),
            pltpu.VMEM((EB,), f32),
            pltpu.VMEM_SHARED((n,), f32),
        ],
    )(src, dst, asrc.reshape(n), adst.reshape(n), cvec, zd)

    num2 = pl.kernel(
        _sc_row_body,
        out_type=jax.ShapeDtypeStruct((2 * n, HW), f32),
        mesh=mesh,
        compiler_params=scp,
        scratch_types=[
            pltpu.VMEM((2, EB), jnp.int32),
            pltpu.VMEM((2, NSUB, SUB), jnp.int32),
            pltpu.VMEM((2, NSUB, SUB), jnp.int32),
            pltpu.VMEM((2, EB), f32),
            pltpu.VMEM((2, EB, HW), f32),
            pltpu.VMEM_SHARED((n, HW), f32),
            pltpu.SemaphoreType.DMA,
            pltpu.SemaphoreType.DMA,
            pltpu.SemaphoreType.DMA,
            pltpu.SemaphoreType.DMA,
            pltpu.SemaphoreType.DMA,
            pltpu.SemaphoreType.DMA,
        ],
    )(src, dst, wvals, xp2, zn)

    h_next, c_next = pl.pallas_call(
        _finish_body,
        grid=(g,),
        in_specs=[
            pl.BlockSpec((BN, HW), lambda i: (i, 0)),
            pl.BlockSpec((BN, HW), lambda i: (g + i, 0)),
            pl.BlockSpec((BN, 1), lambda i: (i, 0)),
            pl.BlockSpec((BN, 1), lambda i: (g + i, 0)),
            pl.BlockSpec((BN, dh), lambda i: (i, 0)),
            pl.BlockSpec((BN, 1), lambda i: (i, 0)),
            pl.BlockSpec((BN, 1), lambda i: (i, 0)),
            pl.BlockSpec((1, 1), lambda i: (0, 0)),
            pl.BlockSpec((BN, dh), lambda i: (i, 0)),
            pl.BlockSpec((BN, dh), lambda i: (i, 0)),
            pl.BlockSpec((4 * dh, dh), lambda i: (0, 0)),
            pl.BlockSpec((4 * dh, dh), lambda i: (0, 0)),
            pl.BlockSpec((1, 4 * dh), lambda i: (0, 0)),
            pl.BlockSpec((1, 4 * dh), lambda i: (0, 0)),
            pl.BlockSpec((1, dh), lambda i: (0, 0)),
        ],
        out_specs=[
            pl.BlockSpec((BN, dh), lambda i: (i, 0)),
            pl.BlockSpec((BN, dh), lambda i: (i, 0)),
        ],
        out_shape=[
            jax.ShapeDtypeStruct((n, dh), f32),
            jax.ShapeDtypeStruct((n, dh), f32),
        ],
    )(num2, num2, den2.reshape(2 * n, 1), den2.reshape(2 * n, 1), xp, asrc,
      adst, cmax, h_prev, c_prev, W_ih, W_hh, b_ih.reshape(1, 4 * dh),
      b_hh.reshape(1, 4 * dh), b_gat.reshape(1, dh))
    return (h_next, c_next)


# speedup vs baseline: 39.1764x; 1.2102x over previous
import functools

import jax
import jax.numpy as jnp
from jax import lax
from jax.experimental import pallas as pl
from jax.experimental.pallas import tpu as pltpu
from jax.experimental.pallas import tpu_sc as plsc

NC = 2
NS = 16
LN = 16
HW = 32

BN = 2000
EB = 400
SUB = 80
NSUB = EB // SUB


def _prep_body(x_ref, h_ref, w1_ref, w2_ref, as_ref, ad_ref,
               xp_ref, asrc_ref, adst_ref, cmax_ref, acc_ref):
    i = pl.program_id(0)
    xp = (jnp.dot(x_ref[...], w1_ref[...], preferred_element_type=jnp.float32)
          + jnp.dot(h_ref[...], w2_ref[...], preferred_element_type=jnp.float32))
    a_s = jnp.dot(xp, as_ref[...], preferred_element_type=jnp.float32)
    a_d = jnp.dot(xp, ad_ref[...], preferred_element_type=jnp.float32)
    xp_ref[...] = xp
    asrc_ref[...] = a_s
    adst_ref[...] = a_d

    @pl.when(i == 0)
    def _():
        acc_ref[0] = jnp.float32(-1e30)
        acc_ref[1] = jnp.float32(-1e30)

    acc_ref[0] = jnp.maximum(acc_ref[0], jnp.max(a_s))
    acc_ref[1] = jnp.maximum(acc_ref[1], jnp.max(a_d))
    cmax_ref[...] = jnp.full((1, 1), jnp.maximum(acc_ref[0] + acc_ref[1],
                                                 jnp.float32(0.0)), jnp.float32)


def _sc_scalar_body(src_hbm, dst_hbm, asrc_hbm, adst_hbm, c_hbm, zd_hbm,
                    w_hbm, den_hbm,
                    asrc_tbl, adst_tbl, cbuf, srcbuf, dst2, sdst2, wbuf,
                    den_acc, sem_i0, sem_i1, sem_o0, sem_o1):
    n = asrc_tbl.shape[0]
    ept = src_hbm.shape[0] // NS
    nch = ept // EB
    cid = lax.axis_index("c")
    tid = lax.axis_index("s")
    sem_i = (sem_i0, sem_i1)
    sem_o = (sem_o0, sem_o1)

    def idx_trips(k, b):
        e0 = tid * ept + k * EB
        t = [(src_hbm.at[pl.ds(e0, EB)], srcbuf.at[b])]
        t += [(dst_hbm.at[pl.ds(e0 + j * SUB, SUB)], dst2.at[b, j])
              for j in range(NSUB)]
        return t

    def issue_idx(k, b):
        for s, d in idx_trips(k, b):
            pltpu.async_copy(s, d, sem_i[b])

    def wait_idx(k, b):
        for s, d in idx_trips(k, b):
            pltpu.make_async_copy(s, d, sem_i[b]).wait()

    def drain_out(b):
        for j in range(NSUB):
            pltpu.make_async_copy(wbuf.at[b, pl.ds(j * SUB, SUB)],
                                  den_acc.at[sdst2.at[b, j]], sem_o[b]).wait()

    def process(k, b):
        @pl.when(k >= 2)
        def _():
            drain_out(b)

        wait_idx(k, b)
        e0 = tid * ept + k * EB
        for j in range(NSUB):
            for m in range(SUB // LN):
                off = j * SUB + m * LN
                sv = srcbuf[b, pl.ds(off, LN)]
                dv = dst2[b, j, pl.ds(m * LN, LN)]
                av = plsc.load_gather(asrc_tbl, [sv])
                bv = plsc.load_gather(adst_tbl, [dv])
                e = av + bv
                e = jnp.where(e >= 0, e, jnp.float32(0.2) * e)
                wbuf[b, pl.ds(off, LN)] = jnp.exp(e - cvec)
                sdst2[b, j, pl.ds(m * LN, LN)] = dv
        @pl.when(cid == 0)
        def _():
            pltpu.sync_copy(wbuf.at[b], w_hbm.at[pl.ds(e0, EB)])

        for j in range(NSUB):
            pltpu.async_copy(wbuf.at[b, pl.ds(j * SUB, SUB)],
                             den_acc.at[sdst2.at[b, j]], sem_o[b], add=True)

        @pl.when(k + 2 < nch)
        def _():
            issue_idx(k + 2, b)

    pltpu.sync_copy(asrc_hbm, asrc_tbl)
    pltpu.sync_copy(adst_hbm, adst_tbl)
    pltpu.sync_copy(c_hbm, cbuf)
    cvec = cbuf[...]

    issue_idx(0, 0)
    issue_idx(1, 1)

    @pl.when(tid == 0)
    def _():
        pltpu.sync_copy(zd_hbm, den_acc)

    plsc.subcore_barrier()

    def loop2(k2, c):
        process(2 * k2, 0)
        process(2 * k2 + 1, 1)
        return c

    lax.fori_loop(0, nch // 2, loop2, 0)
    for k in range(2 * (nch // 2), nch):
        process(jnp.int32(k), k % 2)
    for b in range(2):
        drain_out(b)
    plsc.subcore_barrier()

    @pl.when((tid == 0) & (cid == 0))
    def _():
        pltpu.sync_copy(den_acc, den_hbm)


def _sc_row_body(src_hbm, dst_hbm, w_hbm, xp2_hbm, zn_hbm, num_hbm,
                 srcbuf, dst2, sdst2, wbuf, rows, num_acc,
                 sem_i0, sem_i1, sem_g0, sem_g1, sem_s0, sem_s1):
    n = num_acc.shape[0]
    ept = src_hbm.shape[0] // NS
    nch = ept // EB
    cid = lax.axis_index("c")
    tid = lax.axis_index("s")
    coff = (cid * n).astype(jnp.int32)
    sem_i = (sem_i0, sem_i1)
    sem_g = (sem_g0, sem_g1)
    sem_s = (sem_s0, sem_s1)

    rpt8 = ((n // NS + 7) // 8) * 8
    last = n - (NS - 1) * rpt8
    assert last > 0 and last % 8 == 0

    def idx_trips(k, b):
        e0 = tid * ept + k * EB
        t = [(src_hbm.at[pl.ds(e0, EB)], srcbuf.at[b]),
             (w_hbm.at[pl.ds(e0, EB)], wbuf.at[b])]
        t += [(dst_hbm.at[pl.ds(e0 + j * SUB, SUB)], dst2.at[b, j])
              for j in range(NSUB)]
        return t

    def issue_idx(k, b):
        for s, d in idx_trips(k, b):
            pltpu.async_copy(s, d, sem_i[b])

    def wait_idx(k, b):
        for s, d in idx_trips(k, b):
            pltpu.make_async_copy(s, d, sem_i[b]).wait()

    def drain_scat(b):
        for j in range(NSUB):
            pltpu.make_async_copy(rows.at[b, pl.ds(j * SUB, SUB)],
                                  num_acc.at[sdst2.at[b, j]], sem_s[b]).wait()

    def fire_gather(j, b):
        pltpu.async_copy(xp2_hbm.at[srcbuf.at[b, pl.ds(j * SUB, SUB)]],
                         rows.at[b, pl.ds(j * SUB, SUB)], sem_g[j % 2])

    def drain_gather(j, b):
        pltpu.make_async_copy(xp2_hbm.at[srcbuf.at[b, pl.ds(j * SUB, SUB)]],
                              rows.at[b, pl.ds(j * SUB, SUB)],
                              sem_g[j % 2]).wait()

    def process(k, b):
        @pl.when(k >= 2)
        def _():
            drain_scat(b)

        wait_idx(k, b)

        def shift(gi, c):
            off = gi * LN
            srcbuf[b, pl.ds(off, LN)] = srcbuf[b, pl.ds(off, LN)] + coff
            return c

        lax.fori_loop(0, EB // LN, shift, 0)
        for j in range(NSUB):
            for m in range(SUB // LN):
                sdst2[b, j, pl.ds(m * LN, LN)] = dst2[b, j, pl.ds(m * LN, LN)]

        def scale(gi, c):
            wv = wbuf[b, pl.ds(gi * LN, LN)]
            for l in range(LN):
                i = gi * LN + l
                wl = jnp.full((LN,), wv[l], jnp.float32)
                rows[b, i, pl.ds(0, LN)] = rows[b, i, pl.ds(0, LN)] * wl
                rows[b, i, pl.ds(LN, LN)] = rows[b, i, pl.ds(LN, LN)] * wl
            return c

        fire_gather(0, b)
        for j in range(NSUB):
            if j + 1 < NSUB:
                fire_gather(j + 1, b)
            drain_gather(j, b)
            lax.fori_loop(j * (SUB // LN), (j + 1) * (SUB // LN), scale, 0)
            pltpu.async_copy(rows.at[b, pl.ds(j * SUB, SUB)],
                             num_acc.at[sdst2.at[b, j]], sem_s[b], add=True)

        @pl.when(k + 2 < nch)
        def _():
            issue_idx(k + 2, b)

    issue_idx(0, 0)
    issue_idx(1, 1)

    @pl.when(tid < NS - 1)
    def _():
        pltpu.sync_copy(zn_hbm.at[pl.ds(tid * rpt8, rpt8)],
                        num_acc.at[pl.ds(tid * rpt8, rpt8)])

    @pl.when(tid == NS - 1)
    def _():
        pltpu.sync_copy(zn_hbm.at[pl.ds((NS - 1) * rpt8, last)],
                        num_acc.at[pl.ds((NS - 1) * rpt8, last)])

    plsc.subcore_barrier()

    def loop2(k2, c):
        process(2 * k2, 0)
        process(2 * k2 + 1, 1)
        return c

    lax.fori_loop(0, nch // 2, loop2, 0)
    for k in range(2 * (nch // 2), nch):
        process(jnp.int32(k), k % 2)
    for b in range(2):
        drain_scat(b)
    plsc.subcore_barrier()

    @pl.when(tid < NS - 1)
    def _():
        pltpu.sync_copy(num_acc.at[pl.ds(tid * rpt8, rpt8)],
                        num_hbm.at[pl.ds(cid * n + tid * rpt8, rpt8)])

    @pl.when(tid == NS - 1)
    def _():
        pltpu.sync_copy(num_acc.at[pl.ds((NS - 1) * rpt8, last)],
                        num_hbm.at[pl.ds(cid * n + (NS - 1) * rpt8, last)])


def _finish_body(numl_ref, numr_ref, den_ref, xp_ref, asrc_ref,
                 adst_ref, cmax_ref, h_ref, c_ref, wih_ref, whh_ref, bih_ref,
                 bhh_ref, bgat_ref, h_out, c_out):
    a = asrc_ref[...] + adst_ref[...]
    a = jnp.where(a >= 0, a, jnp.float32(0.2) * a)
    ws = jnp.exp(a - cmax_ref[...])
    num = jnp.concatenate([numl_ref[...], numr_ref[...]], axis=1)
    den = den_ref[...] + ws + jnp.float32(1e-16)
    hagg = (num + ws * xp_ref[...]) / den + bgat_ref[...]
    dn = (((1,), (1,)), ((), ()))
    gates = (lax.dot_general(hagg, wih_ref[...], dn,
                             preferred_element_type=jnp.float32)
             + lax.dot_general(h_ref[...], whh_ref[...], dn,
                               preferred_element_type=jnp.float32)
             + bih_ref[...] + bhh_ref[...])
    dh = h_ref.shape[1]
    ig = jax.nn.sigmoid(gates[:, 0:dh])
    fg = jax.nn.sigmoid(gates[:, dh:2 * dh])
    gg = jnp.tanh(gates[:, 2 * dh:3 * dh])
    og = jax.nn.sigmoid(gates[:, 3 * dh:4 * dh])
    cn = fg * c_ref[...] + ig * gg
    h_out[...] = og * jnp.tanh(cn)
    c_out[...] = cn


def kernel(x, h_prev, c_prev, edge_index, W_gat, att_src, att_dst, b_gat,
           W_ih, b_ih, W_hh, b_hh):
    n, d_in = x.shape
    dh = h_prev.shape[1]
    e = edge_index.shape[1]
    g = n // BN
    assert n % BN == 0 and n % NS == 0 and dh == 2 * HW
    assert e % (NS * EB) == 0

    f32 = jnp.float32
    w1 = W_gat[:d_in]
    w2 = W_gat[d_in:]
    as2 = att_src.reshape(dh, 1)
    ad2 = att_dst.reshape(dh, 1)

    xp, asrc, adst, cmax = pl.pallas_call(
        _prep_body,
        grid=(g,),
        in_specs=[
            pl.BlockSpec((BN, d_in), lambda i: (i, 0)),
            pl.BlockSpec((BN, dh), lambda i: (i, 0)),
            pl.BlockSpec((d_in, dh), lambda i: (0, 0)),
            pl.BlockSpec((dh, dh), lambda i: (0, 0)),
            pl.BlockSpec((dh, 1), lambda i: (0, 0)),
            pl.BlockSpec((dh, 1), lambda i: (0, 0)),
        ],
        out_specs=[
            pl.BlockSpec((BN, dh), lambda i: (i, 0)),
            pl.BlockSpec((BN, 1), lambda i: (i, 0)),
            pl.BlockSpec((BN, 1), lambda i: (i, 0)),
            pl.BlockSpec((1, 1), lambda i: (0, 0)),
        ],
        out_shape=[
            jax.ShapeDtypeStruct((n, dh), f32),
            jax.ShapeDtypeStruct((n, 1), f32),
            jax.ShapeDtypeStruct((n, 1), f32),
            jax.ShapeDtypeStruct((1, 1), f32),
        ],
        scratch_shapes=[pltpu.SMEM((2,), f32)],
    )(x, h_prev, w1, w2, as2, ad2)

    xp2 = jnp.concatenate([xp[:, 0:HW], xp[:, HW:2 * HW]], axis=0)
    src = edge_index[0]
    dst = edge_index[1]
    cvec = jnp.broadcast_to(cmax.reshape(1), (LN,))
    zn = jnp.zeros((n, HW), f32)
    zd = jnp.zeros((n,), f32)

    mesh = plsc.VectorSubcoreMesh(core_axis_name="c", subcore_axis_name="s",
                                  num_cores=NC, num_subcores=NS)
    scp = pltpu.CompilerParams(needs_layout_passes=False,
                               use_tc_tiling_on_sc=False)
    wvals, den = pl.kernel(
        _sc_scalar_body,
        out_type=[
            jax.ShapeDtypeStruct((e,), f32),
            jax.ShapeDtypeStruct((n,), f32),
        ],
        mesh=mesh,
        compiler_params=scp,
        scratch_types=[
            pltpu.VMEM((n,), f32),
            pltpu.VMEM((n,), f32),
            pltpu.VMEM((LN,), f32),
            pltpu.VMEM((2, EB), jnp.int32),
            pltpu.VMEM((2, NSUB, SUB), jnp.int32),
            pltpu.VMEM((2, NSUB, SUB), jnp.int32),
            pltpu.VMEM((2, EB), f32),
            pltpu.VMEM_SHARED((n,), f32),
            pltpu.SemaphoreType.DMA,
            pltpu.SemaphoreType.DMA,
            pltpu.SemaphoreType.DMA,
            pltpu.SemaphoreType.DMA,
        ],
    )(src, dst, asrc.reshape(n), adst.reshape(n), cvec, zd)

    num2 = pl.kernel(
        _sc_row_body,
        out_type=jax.ShapeDtypeStruct((2 * n, HW), f32),
        mesh=mesh,
        compiler_params=scp,
        scratch_types=[
            pltpu.VMEM((2, EB), jnp.int32),
            pltpu.VMEM((2, NSUB, SUB), jnp.int32),
            pltpu.VMEM((2, NSUB, SUB), jnp.int32),
            pltpu.VMEM((2, EB), f32),
            pltpu.VMEM((2, EB, HW), f32),
            pltpu.VMEM_SHARED((n, HW), f32),
            pltpu.SemaphoreType.DMA,
            pltpu.SemaphoreType.DMA,
            pltpu.SemaphoreType.DMA,
            pltpu.SemaphoreType.DMA,
            pltpu.SemaphoreType.DMA,
            pltpu.SemaphoreType.DMA,
        ],
    )(src, dst, wvals, xp2, zn)

    h_next, c_next = pl.pallas_call(
        _finish_body,
        grid=(g,),
        in_specs=[
            pl.BlockSpec((BN, HW), lambda i: (i, 0)),
            pl.BlockSpec((BN, HW), lambda i: (g + i, 0)),
            pl.BlockSpec((BN, 1), lambda i: (i, 0)),
            pl.BlockSpec((BN, dh), lambda i: (i, 0)),
            pl.BlockSpec((BN, 1), lambda i: (i, 0)),
            pl.BlockSpec((BN, 1), lambda i: (i, 0)),
            pl.BlockSpec((1, 1), lambda i: (0, 0)),
            pl.BlockSpec((BN, dh), lambda i: (i, 0)),
            pl.BlockSpec((BN, dh), lambda i: (i, 0)),
            pl.BlockSpec((4 * dh, dh), lambda i: (0, 0)),
            pl.BlockSpec((4 * dh, dh), lambda i: (0, 0)),
            pl.BlockSpec((1, 4 * dh), lambda i: (0, 0)),
            pl.BlockSpec((1, 4 * dh), lambda i: (0, 0)),
            pl.BlockSpec((1, dh), lambda i: (0, 0)),
        ],
        out_specs=[
            pl.BlockSpec((BN, dh), lambda i: (i, 0)),
            pl.BlockSpec((BN, dh), lambda i: (i, 0)),
        ],
        out_shape=[
            jax.ShapeDtypeStruct((n, dh), f32),
            jax.ShapeDtypeStruct((n, dh), f32),
        ],
    )(num2, num2, den.reshape(n, 1), xp, asrc,
      adst, cmax, h_prev, c_prev, W_ih, W_hh, b_ih.reshape(1, 4 * dh),
      b_hh.reshape(1, 4 * dh), b_gat.reshape(1, dh))
    return (h_next, c_next)


# speedup vs baseline: 41.1396x; 1.0501x over previous
import functools

import jax
import jax.numpy as jnp
from jax import lax
from jax.experimental import pallas as pl
from jax.experimental.pallas import tpu as pltpu
from jax.experimental.pallas import tpu_sc as plsc

NC = 2
NS = 16
LN = 16
HW = 32

BN = 2000
EB = 400
SUB = 80
NSUB = EB // SUB


def _prep_body(x_ref, h_ref, w1_ref, w2_ref, as_ref, ad_ref,
               xp_ref, asrc_ref, adst_ref, cmax_ref, acc_ref):
    i = pl.program_id(0)
    xp = (jnp.dot(x_ref[...], w1_ref[...], preferred_element_type=jnp.float32)
          + jnp.dot(h_ref[...], w2_ref[...], preferred_element_type=jnp.float32))
    a_s = jnp.dot(xp, as_ref[...], preferred_element_type=jnp.float32)
    a_d = jnp.dot(xp, ad_ref[...], preferred_element_type=jnp.float32)
    xp_ref[...] = xp
    asrc_ref[...] = a_s
    adst_ref[...] = a_d

    @pl.when(i == 0)
    def _():
        acc_ref[0] = jnp.float32(-1e30)
        acc_ref[1] = jnp.float32(-1e30)

    acc_ref[0] = jnp.maximum(acc_ref[0], jnp.max(a_s))
    acc_ref[1] = jnp.maximum(acc_ref[1], jnp.max(a_d))
    cmax_ref[...] = jnp.full((1, 1), jnp.maximum(acc_ref[0] + acc_ref[1],
                                                 jnp.float32(0.0)), jnp.float32)


def _sc_scalar_body(src_hbm, dst_hbm, asrc_hbm, adst_hbm, c_hbm, zd_hbm,
                    w_hbm, den_hbm,
                    asrc_tbl, adst_tbl, cbuf, srcbuf, dst2, sdst2, wbuf,
                    den_acc, sem_i0, sem_i1, sem_o0, sem_o1):
    n = asrc_tbl.shape[0]
    ept = src_hbm.shape[0] // NS
    nch = ept // EB
    cid = lax.axis_index("c")
    tid = lax.axis_index("s")
    sem_i = (sem_i0, sem_i1)
    sem_o = (sem_o0, sem_o1)

    def idx_trips(k, b):
        e0 = tid * ept + k * EB
        t = [(src_hbm.at[pl.ds(e0, EB)], srcbuf.at[b])]
        t += [(dst_hbm.at[pl.ds(e0 + j * SUB, SUB)], dst2.at[b, j])
              for j in range(NSUB)]
        return t

    def issue_idx(k, b):
        for s, d in idx_trips(k, b):
            pltpu.async_copy(s, d, sem_i[b])

    def wait_idx(k, b):
        for s, d in idx_trips(k, b):
            pltpu.make_async_copy(s, d, sem_i[b]).wait()

    def drain_out(b):
        for j in range(NSUB):
            pltpu.make_async_copy(wbuf.at[b, pl.ds(j * SUB, SUB)],
                                  den_acc.at[sdst2.at[b, j]], sem_o[b]).wait()

    def process(k, b):
        @pl.when(k >= 2)
        def _():
            drain_out(b)

        wait_idx(k, b)
        e0 = tid * ept + k * EB
        for j in range(NSUB):
            for m in range(SUB // LN):
                off = j * SUB + m * LN
                sv = srcbuf[b, pl.ds(off, LN)]
                dv = dst2[b, j, pl.ds(m * LN, LN)]
                av = plsc.load_gather(asrc_tbl, [sv])
                bv = plsc.load_gather(adst_tbl, [dv])
                e = av + bv
                e = jnp.where(e >= 0, e, jnp.float32(0.2) * e)
                wbuf[b, pl.ds(off, LN)] = jnp.exp(e - cvec)
                sdst2[b, j, pl.ds(m * LN, LN)] = dv
        @pl.when(k % 2 == cid)
        def _():
            pltpu.sync_copy(wbuf.at[b], w_hbm.at[pl.ds(e0, EB)])

        for j in range(NSUB):
            pltpu.async_copy(wbuf.at[b, pl.ds(j * SUB, SUB)],
                             den_acc.at[sdst2.at[b, j]], sem_o[b], add=True)

        @pl.when(k + 2 < nch)
        def _():
            issue_idx(k + 2, b)

    pltpu.sync_copy(asrc_hbm, asrc_tbl)
    pltpu.sync_copy(adst_hbm, adst_tbl)
    pltpu.sync_copy(c_hbm, cbuf)
    cvec = cbuf[...]

    issue_idx(0, 0)
    issue_idx(1, 1)

    @pl.when(tid == 0)
    def _():
        pltpu.sync_copy(zd_hbm, den_acc)

    plsc.subcore_barrier()

    def loop2(k2, c):
        process(2 * k2, 0)
        process(2 * k2 + 1, 1)
        return c

    lax.fori_loop(0, nch // 2, loop2, 0)
    for k in range(2 * (nch // 2), nch):
        process(jnp.int32(k), k % 2)
    for b in range(2):
        drain_out(b)
    plsc.subcore_barrier()

    @pl.when((tid == 0) & (cid == 0))
    def _():
        pltpu.sync_copy(den_acc, den_hbm)


def _sc_row_body(src_hbm, dst_hbm, w_hbm, xp2_hbm, num_hbm,
                 srcbuf, dst2, sdst2, wbuf, rows, num_acc,
                 sem_i0, sem_i1, sem_g0, sem_g1, sem_s0, sem_s1):
    n = num_acc.shape[0]
    ept = src_hbm.shape[0] // NS
    nch = ept // EB
    cid = lax.axis_index("c")
    tid = lax.axis_index("s")
    coff = (cid * n).astype(jnp.int32)
    sem_i = (sem_i0, sem_i1)
    sem_g = (sem_g0, sem_g1)
    sem_s = (sem_s0, sem_s1)

    rpt8 = ((n // NS + 7) // 8) * 8
    last = n - (NS - 1) * rpt8
    assert last > 0 and last % 8 == 0

    def idx_trips(k, b):
        e0 = tid * ept + k * EB
        t = [(src_hbm.at[pl.ds(e0, EB)], srcbuf.at[b]),
             (w_hbm.at[pl.ds(e0, EB)], wbuf.at[b])]
        t += [(dst_hbm.at[pl.ds(e0 + j * SUB, SUB)], dst2.at[b, j])
              for j in range(NSUB)]
        return t

    def issue_idx(k, b):
        for s, d in idx_trips(k, b):
            pltpu.async_copy(s, d, sem_i[b])

    def wait_idx(k, b):
        for s, d in idx_trips(k, b):
            pltpu.make_async_copy(s, d, sem_i[b]).wait()

    def drain_scat(b):
        for j in range(NSUB):
            pltpu.make_async_copy(rows.at[b, pl.ds(j * SUB, SUB)],
                                  num_acc.at[sdst2.at[b, j]], sem_s[b]).wait()

    def fire_gather(j, b):
        pltpu.async_copy(xp2_hbm.at[srcbuf.at[b, pl.ds(j * SUB, SUB)]],
                         rows.at[b, pl.ds(j * SUB, SUB)], sem_g[j % 2])

    def drain_gather(j, b):
        pltpu.make_async_copy(xp2_hbm.at[srcbuf.at[b, pl.ds(j * SUB, SUB)]],
                              rows.at[b, pl.ds(j * SUB, SUB)],
                              sem_g[j % 2]).wait()

    def process(k, b):
        @pl.when(k >= 2)
        def _():
            drain_scat(b)

        wait_idx(k, b)

        def shift(gi, c):
            off = gi * LN
            srcbuf[b, pl.ds(off, LN)] = srcbuf[b, pl.ds(off, LN)] + coff
            return c

        lax.fori_loop(0, EB // LN, shift, 0)
        for j in range(NSUB):
            for m in range(SUB // LN):
                sdst2[b, j, pl.ds(m * LN, LN)] = dst2[b, j, pl.ds(m * LN, LN)]

        def scale(gi, c):
            wv = wbuf[b, pl.ds(gi * LN, LN)]
            for l in range(LN):
                i = gi * LN + l
                wl = jnp.full((LN,), wv[l], jnp.float32)
                rows[b, i, pl.ds(0, LN)] = rows[b, i, pl.ds(0, LN)] * wl
                rows[b, i, pl.ds(LN, LN)] = rows[b, i, pl.ds(LN, LN)] * wl
            return c

        fire_gather(0, b)
        fire_gather(1, b)
        for j in range(NSUB):
            drain_gather(j, b)
            if j + 2 < NSUB:
                fire_gather(j + 2, b)
            lax.fori_loop(j * (SUB // LN), (j + 1) * (SUB // LN), scale, 0)
            pltpu.async_copy(rows.at[b, pl.ds(j * SUB, SUB)],
                             num_acc.at[sdst2.at[b, j]], sem_s[b], add=True)

        @pl.when(k + 2 < nch)
        def _():
            issue_idx(k + 2, b)

    issue_idx(0, 0)
    issue_idx(1, 1)

    def zrow(i, c):
        rows[0, i, pl.ds(0, LN)] = jnp.zeros((LN,), jnp.float32)
        rows[0, i, pl.ds(LN, LN)] = jnp.zeros((LN,), jnp.float32)
        return c

    lax.fori_loop(0, EB, zrow, 0)
    nfull = rpt8 // EB
    for u in range(nfull):
        pltpu.sync_copy(rows.at[0],
                        num_acc.at[pl.ds(tid * rpt8 + u * EB, EB)])
    r1 = rpt8 - nfull * EB
    r2 = last - nfull * EB
    assert 0 < r2 <= r1 <= EB and r1 % 8 == 0 and r2 % 8 == 0

    @pl.when(tid < NS - 1)
    def _():
        pltpu.sync_copy(rows.at[0, pl.ds(0, r1)],
                        num_acc.at[pl.ds(tid * rpt8 + nfull * EB, r1)])

    @pl.when(tid == NS - 1)
    def _():
        pltpu.sync_copy(rows.at[0, pl.ds(0, r2)],
                        num_acc.at[pl.ds((NS - 1) * rpt8 + nfull * EB, r2)])

    plsc.subcore_barrier()

    def loop2(k2, c):
        process(2 * k2, 0)
        process(2 * k2 + 1, 1)
        return c

    lax.fori_loop(0, nch // 2, loop2, 0)
    for k in range(2 * (nch // 2), nch):
        process(jnp.int32(k), k % 2)
    for b in range(2):
        drain_scat(b)
    plsc.subcore_barrier()

    @pl.when(tid < NS - 1)
    def _():
        pltpu.sync_copy(num_acc.at[pl.ds(tid * rpt8, rpt8)],
                        num_hbm.at[pl.ds(cid * n + tid * rpt8, rpt8)])

    @pl.when(tid == NS - 1)
    def _():
        pltpu.sync_copy(num_acc.at[pl.ds((NS - 1) * rpt8, last)],
                        num_hbm.at[pl.ds(cid * n + (NS - 1) * rpt8, last)])


def _finish_body(numl_ref, numr_ref, den_ref, xp_ref, asrc_ref,
                 adst_ref, cmax_ref, h_ref, c_ref, wih_ref, whh_ref, bih_ref,
                 bhh_ref, bgat_ref, h_out, c_out):
    a = asrc_ref[...] + adst_ref[...]
    a = jnp.where(a >= 0, a, jnp.float32(0.2) * a)
    ws = jnp.exp(a - cmax_ref[...])
    num = jnp.concatenate([numl_ref[...], numr_ref[...]], axis=1)
    den = den_ref[...] + ws + jnp.float32(1e-16)
    hagg = (num + ws * xp_ref[...]) / den + bgat_ref[...]
    dn = (((1,), (1,)), ((), ()))
    gates = (lax.dot_general(hagg, wih_ref[...], dn,
                             preferred_element_type=jnp.float32)
             + lax.dot_general(h_ref[...], whh_ref[...], dn,
                               preferred_element_type=jnp.float32)
             + bih_ref[...] + bhh_ref[...])
    dh = h_ref.shape[1]
    ig = jax.nn.sigmoid(gates[:, 0:dh])
    fg = jax.nn.sigmoid(gates[:, dh:2 * dh])
    gg = jnp.tanh(gates[:, 2 * dh:3 * dh])
    og = jax.nn.sigmoid(gates[:, 3 * dh:4 * dh])
    cn = fg * c_ref[...] + ig * gg
    h_out[...] = og * jnp.tanh(cn)
    c_out[...] = cn


def kernel(x, h_prev, c_prev, edge_index, W_gat, att_src, att_dst, b_gat,
           W_ih, b_ih, W_hh, b_hh):
    n, d_in = x.shape
    dh = h_prev.shape[1]
    e = edge_index.shape[1]
    g = n // BN
    assert n % BN == 0 and n % NS == 0 and dh == 2 * HW
    assert e % (NS * EB) == 0

    f32 = jnp.float32
    w1 = W_gat[:d_in]
    w2 = W_gat[d_in:]
    as2 = att_src.reshape(dh, 1)
    ad2 = att_dst.reshape(dh, 1)

    xp, asrc, adst, cmax = pl.pallas_call(
        _prep_body,
        grid=(g,),
        in_specs=[
            pl.BlockSpec((BN, d_in), lambda i: (i, 0)),
            pl.BlockSpec((BN, dh), lambda i: (i, 0)),
            pl.BlockSpec((d_in, dh), lambda i: (0, 0)),
            pl.BlockSpec((dh, dh), lambda i: (0, 0)),
            pl.BlockSpec((dh, 1), lambda i: (0, 0)),
            pl.BlockSpec((dh, 1), lambda i: (0, 0)),
        ],
        out_specs=[
            pl.BlockSpec((BN, dh), lambda i: (i, 0)),
            pl.BlockSpec((BN, 1), lambda i: (i, 0)),
            pl.BlockSpec((BN, 1), lambda i: (i, 0)),
            pl.BlockSpec((1, 1), lambda i: (0, 0)),
        ],
        out_shape=[
            jax.ShapeDtypeStruct((n, dh), f32),
            jax.ShapeDtypeStruct((n, 1), f32),
            jax.ShapeDtypeStruct((n, 1), f32),
            jax.ShapeDtypeStruct((1, 1), f32),
        ],
        scratch_shapes=[pltpu.SMEM((2,), f32)],
    )(x, h_prev, w1, w2, as2, ad2)

    xp2 = jnp.concatenate([xp[:, 0:HW], xp[:, HW:2 * HW]], axis=0)
    src = edge_index[0]
    dst = edge_index[1]
    cvec = jnp.broadcast_to(cmax.reshape(1), (LN,))
    zd = jnp.zeros((n,), f32)

    mesh = plsc.VectorSubcoreMesh(core_axis_name="c", subcore_axis_name="s",
                                  num_cores=NC, num_subcores=NS)
    scp = pltpu.CompilerParams(needs_layout_passes=False,
                               use_tc_tiling_on_sc=False)
    wvals, den = pl.kernel(
        _sc_scalar_body,
        out_type=[
            jax.ShapeDtypeStruct((e,), f32),
            jax.ShapeDtypeStruct((n,), f32),
        ],
        mesh=mesh,
        compiler_params=scp,
        scratch_types=[
            pltpu.VMEM((n,), f32),
            pltpu.VMEM((n,), f32),
            pltpu.VMEM((LN,), f32),
            pltpu.VMEM((2, EB), jnp.int32),
            pltpu.VMEM((2, NSUB, SUB), jnp.int32),
            pltpu.VMEM((2, NSUB, SUB), jnp.int32),
            pltpu.VMEM((2, EB), f32),
            pltpu.VMEM_SHARED((n,), f32),
            pltpu.SemaphoreType.DMA,
            pltpu.SemaphoreType.DMA,
            pltpu.SemaphoreType.DMA,
            pltpu.SemaphoreType.DMA,
        ],
    )(src, dst, asrc.reshape(n), adst.reshape(n), cvec, zd)

    num2 = pl.kernel(
        _sc_row_body,
        out_type=jax.ShapeDtypeStruct((2 * n, HW), f32),
        mesh=mesh,
        compiler_params=scp,
        scratch_types=[
            pltpu.VMEM((2, EB), jnp.int32),
            pltpu.VMEM((2, NSUB, SUB), jnp.int32),
            pltpu.VMEM((2, NSUB, SUB), jnp.int32),
            pltpu.VMEM((2, EB), f32),
            pltpu.VMEM((2, EB, HW), f32),
            pltpu.VMEM_SHARED((n, HW), f32),
            pltpu.SemaphoreType.DMA,
            pltpu.SemaphoreType.DMA,
            pltpu.SemaphoreType.DMA,
            pltpu.SemaphoreType.DMA,
            pltpu.SemaphoreType.DMA,
            pltpu.SemaphoreType.DMA,
        ],
    )(src, dst, wvals, xp2)

    h_next, c_next = pl.pallas_call(
        _finish_body,
        grid=(g,),
        in_specs=[
            pl.BlockSpec((BN, HW), lambda i: (i, 0)),
            pl.BlockSpec((BN, HW), lambda i: (g + i, 0)),
            pl.BlockSpec((BN, 1), lambda i: (i, 0)),
            pl.BlockSpec((BN, dh), lambda i: (i, 0)),
            pl.BlockSpec((BN, 1), lambda i: (i, 0)),
            pl.BlockSpec((BN, 1), lambda i: (i, 0)),
            pl.BlockSpec((1, 1), lambda i: (0, 0)),
            pl.BlockSpec((BN, dh), lambda i: (i, 0)),
            pl.BlockSpec((BN, dh), lambda i: (i, 0)),
            pl.BlockSpec((4 * dh, dh), lambda i: (0, 0)),
            pl.BlockSpec((4 * dh, dh), lambda i: (0, 0)),
            pl.BlockSpec((1, 4 * dh), lambda i: (0, 0)),
            pl.BlockSpec((1, 4 * dh), lambda i: (0, 0)),
            pl.BlockSpec((1, dh), lambda i: (0, 0)),
        ],
        out_specs=[
            pl.BlockSpec((BN, dh), lambda i: (i, 0)),
            pl.BlockSpec((BN, dh), lambda i: (i, 0)),
        ],
        out_shape=[
            jax.ShapeDtypeStruct((n, dh), f32),
            jax.ShapeDtypeStruct((n, dh), f32),
        ],
    )(num2, num2, den.reshape(n, 1), xp, asrc,
      adst, cmax, h_prev, c_prev, W_ih, W_hh, b_ih.reshape(1, 4 * dh),
      b_hh.reshape(1, 4 * dh), b_gat.reshape(1, dh))
    return (h_next, c_next)
